# baseline (speedup 1.0000x reference)
import functools

import jax
import jax.numpy as jnp
import numpy as np
from jax import lax
from jax.experimental import pallas as pl
from jax.experimental.pallas import tpu as pltpu

_N = 8192
_N0 = 1638
_N1 = 409
_K = 64
_R0 = 0.2
_R1 = 0.4


def _mlp(ps, x):
    for W, b in ps:
        x = jax.nn.relu(x @ W + b)
    return x


def _fps(pos, npoint):
    d = jnp.sum((pos - pos[0]) ** 2, axis=1)
    sel = jnp.zeros((npoint,), jnp.int32)

    def body(i, carry):
        sel, d = carry
        nxt = jnp.argmax(d).astype(jnp.int32)
        sel = sel.at[i].set(nxt)
        d = jnp.minimum(d, jnp.sum((pos - pos[nxt]) ** 2, axis=1))
        return (sel, d)

    sel, _ = lax.fori_loop(1, npoint, body, (sel, d))
    return sel


def _radius_nbrs(pos_src, pos_q, r, k):
    d2 = jnp.sum((pos_q[:, None, :] - pos_src[None, :, :]) ** 2, axis=-1)
    masked = jnp.where(d2 <= r * r, d2, jnp.inf)
    negv, idx = lax.top_k(-masked, k)
    valid = jnp.isfinite(negv)
    return idx, valid


def _point_conv(ps, x_src, pos_src, pos_q, nbr, valid):
    x_j = x_src[nbr]
    rel = pos_src[nbr] - pos_q[:, None, :]
    h = _mlp(ps, jnp.concatenate([x_j, rel], axis=-1))
    h = jnp.where(valid[:, :, None], h, -jnp.inf)
    out = jnp.max(h, axis=1)
    return jnp.where(jnp.isfinite(out), out, 0.0)


def _knn_interp(x_src, pos_src, pos_dst, k):
    d2 = jnp.sum((pos_dst[:, None, :] - pos_src[None, :, :]) ** 2, axis=-1)
    negv, idx = lax.top_k(-d2, k)
    d2k = jnp.maximum(-negv, 1e-16)
    w = 1.0 / d2k
    w = w / jnp.sum(w, axis=1, keepdims=True)
    return jnp.sum(w[:, :, None] * x_src[idx], axis=1)


def _head_body(h_ref, w0, b0, w1, b1, w2, b2, wl0, bl0, wl1, bl1, wo, bo, out_ref):
    h = h_ref[...]
    h = jnp.maximum(jnp.dot(h, w0[...], preferred_element_type=jnp.float32) + b0[...], 0.0)
    h = jnp.maximum(jnp.dot(h, w1[...], preferred_element_type=jnp.float32) + b1[...], 0.0)
    h = jnp.maximum(jnp.dot(h, w2[...], preferred_element_type=jnp.float32) + b2[...], 0.0)
    h = jnp.maximum(jnp.dot(h, wl0[...], preferred_element_type=jnp.float32) + bl0[...], 0.0)
    h = jnp.maximum(jnp.dot(h, wl1[...], preferred_element_type=jnp.float32) + bl1[...], 0.0)
    out_ref[...] = jnp.dot(h, wo[...], preferred_element_type=jnp.float32) + bo[...]


def _head_pallas(h, params):
    n = h.shape[0]
    kin = h.shape[1]
    (w0, b0), (w1, b1), (w2, b2) = params["fp2"]
    (wl0, bl0), (wl1, bl1) = params["lin"]
    (wo, bo) = params["out"][0]
    kp = 136
    h = jnp.pad(h, ((0, 0), (0, kp - kin)))
    w0 = jnp.pad(w0, ((0, kp - kin), (0, 0)))
    wo_p = jnp.pad(wo, ((0, 0), (0, 128 - wo.shape[1])))
    bo_p = jnp.pad(bo, ((0, 128 - bo.shape[0]),))
    blk = 1024
    grid = (n // blk,)
    wspec = lambda shape: pl.BlockSpec(shape, lambda i: (0,) * len(shape))
    out = pl.pallas_call(
        _head_body,
        grid=grid,
        in_specs=[
            pl.BlockSpec((blk, kp), lambda i: (i, 0)),
            wspec((kp, 128)), wspec((128,)),
            wspec((128, 128)), wspec((128,)),
            wspec((128, 128)), wspec((128,)),
            wspec((128, 128)), wspec((128,)),
            wspec((128, 128)), wspec((128,)),
            wspec((128, 128)), wspec((128,)),
        ],
        out_specs=pl.BlockSpec((blk, 128), lambda i: (i, 0)),
        out_shape=jax.ShapeDtypeStruct((n, 128), jnp.float32),
    )(h, w0, b0, w1, b1, w2, b2, wl0, bl0, wl1, bl1, wo_p, bo_p)
    return out[:, :3]


def kernel(x, pos, batch, params):
    idx0 = _fps(pos, _N0)
    pos0 = pos[idx0]
    nbr0, v0 = _radius_nbrs(pos, pos0, _R0, _K)
    x0 = _point_conv(params["sa0"], x, pos, pos0, nbr0, v0)
    idx1 = _fps(pos0, _N1)
    pos1 = pos0[idx1]
    nbr1, v1 = _radius_nbrs(pos0, pos1, _R1, _K)
    x1 = _point_conv(params["sa1"], x0, pos0, pos1, nbr1, v1)
    g = _mlp(params["sa2"], jnp.concatenate([x1, pos1], axis=1))
    xg = jnp.max(g, axis=0, keepdims=True)
    f0 = jnp.broadcast_to(xg, (_N1, xg.shape[1]))
    f0 = _mlp(params["fp0"], jnp.concatenate([f0, x1], axis=1))
    f1 = _knn_interp(f0, pos1, pos0, 3)
    f1 = _mlp(params["fp1"], jnp.concatenate([f1, x0], axis=1))
    f2 = _knn_interp(f1, pos0, pos, 3)
    h = jnp.concatenate([f2, x], axis=1)
    return _head_pallas(h, params)


# speedup vs baseline: 2.6053x; 2.6053x over previous
import functools

import jax
import jax.numpy as jnp
import numpy as np
from jax import lax
from jax.experimental import pallas as pl
from jax.experimental.pallas import tpu as pltpu

_N = 8192
_N0 = 1638
_N1 = 409
_K = 64
_R0 = 0.2
_R1 = 0.4


def _mlp(ps, x):
    for W, b in ps:
        x = jax.nn.relu(x @ W + b)
    return x


_BIG_I32 = 2**31 - 1


def _fps_body(npoint, n_valid, px_ref, py_ref, pz_ref, ox_ref, oy_ref, oz_ref):
    R, C = px_ref.shape
    OR, OC = ox_ref.shape
    px, py, pz = px_ref[...], py_ref[...], pz_ref[...]
    iota = (lax.broadcasted_iota(jnp.int32, (R, C), 0) * C
            + lax.broadcasted_iota(jnp.int32, (R, C), 1))
    oiota = (lax.broadcasted_iota(jnp.int32, (OR, OC), 0) * OC
             + lax.broadcasted_iota(jnp.int32, (OR, OC), 1))
    valid = iota < n_valid
    x0 = jnp.sum(jnp.where(iota == 0, px, 0.0))
    y0 = jnp.sum(jnp.where(iota == 0, py, 0.0))
    z0 = jnp.sum(jnp.where(iota == 0, pz, 0.0))
    dx, dy, dz = px - x0, py - y0, pz - z0
    d = jnp.where(valid, dx * dx + dy * dy + dz * dz, -1.0)
    ox_ref[...] = jnp.where(oiota == 0, x0, 0.0)
    oy_ref[...] = jnp.where(oiota == 0, y0, 0.0)
    oz_ref[...] = jnp.where(oiota == 0, z0, 0.0)

    def body(i, d):
        m = jnp.max(d)
        nxt = jnp.min(jnp.where(d == m, iota, _BIG_I32))
        sel = iota == nxt
        xn = jnp.sum(jnp.where(sel, px, 0.0))
        yn = jnp.sum(jnp.where(sel, py, 0.0))
        zn = jnp.sum(jnp.where(sel, pz, 0.0))
        dx, dy, dz = px - xn, py - yn, pz - zn
        nd = dx * dx + dy * dy + dz * dz
        om = oiota == i
        ox_ref[...] = jnp.where(om, xn, ox_ref[...])
        oy_ref[...] = jnp.where(om, yn, oy_ref[...])
        oz_ref[...] = jnp.where(om, zn, oz_ref[...])
        return jnp.minimum(d, nd)

    lax.fori_loop(1, npoint, body, d)


def _fps_pallas(pos, n_valid, npoint):
    n_pad = ((n_valid + 127) // 128) * 128
    o_pad = ((npoint + 127) // 128) * 128
    p = jnp.pad(pos, ((0, n_pad - n_valid), (0, 0)))
    px = p[:, 0].reshape(n_pad // 128, 128)
    py = p[:, 1].reshape(n_pad // 128, 128)
    pz = p[:, 2].reshape(n_pad // 128, 128)
    oshape = (o_pad // 128, 128)
    ox, oy, oz = pl.pallas_call(
        functools.partial(_fps_body, npoint, n_valid),
        out_shape=[jax.ShapeDtypeStruct(oshape, jnp.float32)] * 3,
    )(px, py, pz)
    out = jnp.stack([ox.reshape(-1), oy.reshape(-1), oz.reshape(-1)], axis=1)
    return out[:npoint]


def _radius_nbrs(pos_src, pos_q, r, k):
    d2 = jnp.sum((pos_q[:, None, :] - pos_src[None, :, :]) ** 2, axis=-1)
    masked = jnp.where(d2 <= r * r, d2, jnp.inf)
    negv, idx = lax.top_k(-masked, k)
    valid = jnp.isfinite(negv)
    return idx, valid


def _point_conv(ps, x_src, pos_src, pos_q, nbr, valid):
    x_j = x_src[nbr]
    rel = pos_src[nbr] - pos_q[:, None, :]
    h = _mlp(ps, jnp.concatenate([x_j, rel], axis=-1))
    h = jnp.where(valid[:, :, None], h, -jnp.inf)
    out = jnp.max(h, axis=1)
    return jnp.where(jnp.isfinite(out), out, 0.0)


def _knn_interp(x_src, pos_src, pos_dst, k):
    d2 = jnp.sum((pos_dst[:, None, :] - pos_src[None, :, :]) ** 2, axis=-1)
    negv, idx = lax.top_k(-d2, k)
    d2k = jnp.maximum(-negv, 1e-16)
    w = 1.0 / d2k
    w = w / jnp.sum(w, axis=1, keepdims=True)
    return jnp.sum(w[:, :, None] * x_src[idx], axis=1)


def _head_body(h_ref, w0, b0, w1, b1, w2, b2, wl0, bl0, wl1, bl1, wo, bo, out_ref):
    h = h_ref[...]
    h = jnp.maximum(jnp.dot(h, w0[...], preferred_element_type=jnp.float32) + b0[...], 0.0)
    h = jnp.maximum(jnp.dot(h, w1[...], preferred_element_type=jnp.float32) + b1[...], 0.0)
    h = jnp.maximum(jnp.dot(h, w2[...], preferred_element_type=jnp.float32) + b2[...], 0.0)
    h = jnp.maximum(jnp.dot(h, wl0[...], preferred_element_type=jnp.float32) + bl0[...], 0.0)
    h = jnp.maximum(jnp.dot(h, wl1[...], preferred_element_type=jnp.float32) + bl1[...], 0.0)
    out_ref[...] = jnp.dot(h, wo[...], preferred_element_type=jnp.float32) + bo[...]


def _head_pallas(h, params):
    n = h.shape[0]
    kin = h.shape[1]
    (w0, b0), (w1, b1), (w2, b2) = params["fp2"]
    (wl0, bl0), (wl1, bl1) = params["lin"]
    (wo, bo) = params["out"][0]
    kp = 136
    h = jnp.pad(h, ((0, 0), (0, kp - kin)))
    w0 = jnp.pad(w0, ((0, kp - kin), (0, 0)))
    wo_p = jnp.pad(wo, ((0, 0), (0, 128 - wo.shape[1])))
    bo_p = jnp.pad(bo, ((0, 128 - bo.shape[0]),))
    blk = 1024
    grid = (n // blk,)
    wspec = lambda shape: pl.BlockSpec(shape, lambda i: (0,) * len(shape))
    out = pl.pallas_call(
        _head_body,
        grid=grid,
        in_specs=[
            pl.BlockSpec((blk, kp), lambda i: (i, 0)),
            wspec((kp, 128)), wspec((128,)),
            wspec((128, 128)), wspec((128,)),
            wspec((128, 128)), wspec((128,)),
            wspec((128, 128)), wspec((128,)),
            wspec((128, 128)), wspec((128,)),
            wspec((128, 128)), wspec((128,)),
        ],
        out_specs=pl.BlockSpec((blk, 128), lambda i: (i, 0)),
        out_shape=jax.ShapeDtypeStruct((n, 128), jnp.float32),
    )(h, w0, b0, w1, b1, w2, b2, wl0, bl0, wl1, bl1, wo_p, bo_p)
    return out[:, :3]


def kernel(x, pos, batch, params):
    pos0 = _fps_pallas(pos, _N, _N0)
    nbr0, v0 = _radius_nbrs(pos, pos0, _R0, _K)
    x0 = _point_conv(params["sa0"], x, pos, pos0, nbr0, v0)
    pos1 = _fps_pallas(pos0, _N0, _N1)
    nbr1, v1 = _radius_nbrs(pos0, pos1, _R1, _K)
    x1 = _point_conv(params["sa1"], x0, pos0, pos1, nbr1, v1)
    g = _mlp(params["sa2"], jnp.concatenate([x1, pos1], axis=1))
    xg = jnp.max(g, axis=0, keepdims=True)
    f0 = jnp.broadcast_to(xg, (_N1, xg.shape[1]))
    f0 = _mlp(params["fp0"], jnp.concatenate([f0, x1], axis=1))
    f1 = _knn_interp(f0, pos1, pos0, 3)
    f1 = _mlp(params["fp1"], jnp.concatenate([f1, x0], axis=1))
    f2 = _knn_interp(f1, pos0, pos, 3)
    h = jnp.concatenate([f2, x], axis=1)
    return _head_pallas(h, params)


# speedup vs baseline: 2.7647x; 1.0612x over previous
import functools

import jax
import jax.numpy as jnp
import numpy as np
from jax import lax
from jax.experimental import pallas as pl
from jax.experimental.pallas import tpu as pltpu

_N = 8192
_N0 = 1638
_N1 = 409
_K = 64
_R0 = 0.2
_R1 = 0.4


def _mlp(ps, x):
    for W, b in ps:
        x = jax.nn.relu(x @ W + b)
    return x


_BIG_I32 = 2**31 - 1


def _fps_body(npoint, n_valid, px_ref, py_ref, pz_ref, ox_ref, oy_ref, oz_ref):
    R, C = px_ref.shape
    OR, OC = ox_ref.shape
    px, py, pz = px_ref[...], py_ref[...], pz_ref[...]
    iota = (lax.broadcasted_iota(jnp.int32, (R, C), 0) * C
            + lax.broadcasted_iota(jnp.int32, (R, C), 1))
    oiota = (lax.broadcasted_iota(jnp.int32, (OR, OC), 0) * OC
             + lax.broadcasted_iota(jnp.int32, (OR, OC), 1))
    valid = iota < n_valid
    x0 = jnp.sum(jnp.where(iota == 0, px, 0.0))
    y0 = jnp.sum(jnp.where(iota == 0, py, 0.0))
    z0 = jnp.sum(jnp.where(iota == 0, pz, 0.0))
    dx, dy, dz = px - x0, py - y0, pz - z0
    d = jnp.where(valid, dx * dx + dy * dy + dz * dz, -1.0)
    ox_ref[...] = jnp.where(oiota == 0, x0, 0.0)
    oy_ref[...] = jnp.where(oiota == 0, y0, 0.0)
    oz_ref[...] = jnp.where(oiota == 0, z0, 0.0)

    def body(i, d):
        m = jnp.max(d)
        nxt = jnp.min(jnp.where(d == m, iota, _BIG_I32))
        sel = iota == nxt
        xn = jnp.sum(jnp.where(sel, px, 0.0))
        yn = jnp.sum(jnp.where(sel, py, 0.0))
        zn = jnp.sum(jnp.where(sel, pz, 0.0))
        dx, dy, dz = px - xn, py - yn, pz - zn
        nd = dx * dx + dy * dy + dz * dz
        om = oiota == i
        ox_ref[...] = jnp.where(om, xn, ox_ref[...])
        oy_ref[...] = jnp.where(om, yn, oy_ref[...])
        oz_ref[...] = jnp.where(om, zn, oz_ref[...])
        return jnp.minimum(d, nd)

    lax.fori_loop(1, npoint, body, d)


def _fps_pallas(pos, n_valid, npoint):
    n_pad = ((n_valid + 127) // 128) * 128
    o_pad = ((npoint + 127) // 128) * 128
    p = jnp.pad(pos, ((0, n_pad - n_valid), (0, 0)))
    px = p[:, 0].reshape(n_pad // 128, 128)
    py = p[:, 1].reshape(n_pad // 128, 128)
    pz = p[:, 2].reshape(n_pad // 128, 128)
    oshape = (o_pad // 128, 128)
    ox, oy, oz = pl.pallas_call(
        functools.partial(_fps_body, npoint, n_valid),
        out_shape=[jax.ShapeDtypeStruct(oshape, jnp.float32)] * 3,
    )(px, py, pz)
    out = jnp.stack([ox.reshape(-1), oy.reshape(-1), oz.reshape(-1)], axis=1)
    return out[:npoint]


def _radius_nbrs(pos_src, pos_q, r, k):
    d2 = jnp.sum((pos_q[:, None, :] - pos_src[None, :, :]) ** 2, axis=-1)
    masked = jnp.where(d2 <= r * r, d2, jnp.inf)
    negv, idx = lax.top_k(-masked, k)
    valid = jnp.isfinite(negv)
    return idx, valid


def _point_conv(ps, x_src, pos_src, pos_q, nbr, valid):
    x_j = x_src[nbr]
    rel = pos_src[nbr] - pos_q[:, None, :]
    h = _mlp(ps, jnp.concatenate([x_j, rel], axis=-1))
    h = jnp.where(valid[:, :, None], h, -jnp.inf)
    out = jnp.max(h, axis=1)
    return jnp.where(jnp.isfinite(out), out, 0.0)


def _knn3_body(s_valid, q_ref, sx_ref, sy_ref, sz_ref, xs_ref, out_ref):
    Qb = q_ref.shape[0]
    Sp = sx_ref.shape[1]
    qx, qy, qz = q_ref[:, 0:1], q_ref[:, 1:2], q_ref[:, 2:3]
    dx = qx - sx_ref[...]
    dy = qy - sy_ref[...]
    dz = qz - sz_ref[...]
    d2 = dx * dx + dy * dy + dz * dz
    jiota = lax.broadcasted_iota(jnp.int32, (Qb, Sp), 1)
    d2 = jnp.where(jiota < s_valid, d2, jnp.inf)
    Wm = jnp.zeros((Qb, Sp), jnp.float32)
    wsum = jnp.zeros((Qb, 1), jnp.float32)
    ws, sels = [], []
    for _ in range(3):
        m = jnp.min(d2, axis=1, keepdims=True)
        i = jnp.min(jnp.where(d2 == m, jiota, _BIG_I32), axis=1, keepdims=True)
        sel = jiota == i
        d2 = jnp.where(sel, jnp.inf, d2)
        w = 1.0 / jnp.maximum(m, 1e-16)
        wsum = wsum + w
        ws.append(w)
        sels.append(sel)
    for w, sel in zip(ws, sels):
        Wm = Wm + jnp.where(sel, w / wsum, 0.0)
    out_ref[...] = jnp.dot(Wm, xs_ref[...], preferred_element_type=jnp.float32)


def _knn3_pallas(x_src, pos_src, pos_dst):
    S, F = x_src.shape
    Q = pos_dst.shape[0]
    Sp = ((S + 127) // 128) * 128
    Qb = 512
    Qp = ((Q + Qb - 1) // Qb) * Qb
    xs = jnp.pad(x_src, ((0, Sp - S), (0, 0)))
    sp = jnp.pad(pos_src, ((0, Sp - S), (0, 0)))
    sx = sp[:, 0].reshape(1, Sp)
    sy = sp[:, 1].reshape(1, Sp)
    sz = sp[:, 2].reshape(1, Sp)
    qp = jnp.pad(pos_dst, ((0, Qp - Q), (0, 0)))
    out = pl.pallas_call(
        functools.partial(_knn3_body, S),
        grid=(Qp // Qb,),
        in_specs=[
            pl.BlockSpec((Qb, 3), lambda i: (i, 0)),
            pl.BlockSpec((1, Sp), lambda i: (0, 0)),
            pl.BlockSpec((1, Sp), lambda i: (0, 0)),
            pl.BlockSpec((1, Sp), lambda i: (0, 0)),
            pl.BlockSpec((Sp, F), lambda i: (0, 0)),
        ],
        out_specs=pl.BlockSpec((Qb, F), lambda i: (i, 0)),
        out_shape=jax.ShapeDtypeStruct((Qp, F), jnp.float32),
    )(qp, sx, sy, sz, xs)
    return out[:Q]


def _knn_interp(x_src, pos_src, pos_dst, k):
    d2 = jnp.sum((pos_dst[:, None, :] - pos_src[None, :, :]) ** 2, axis=-1)
    negv, idx = lax.top_k(-d2, k)
    d2k = jnp.maximum(-negv, 1e-16)
    w = 1.0 / d2k
    w = w / jnp.sum(w, axis=1, keepdims=True)
    return jnp.sum(w[:, :, None] * x_src[idx], axis=1)


def _head_body(h_ref, w0, b0, w1, b1, w2, b2, wl0, bl0, wl1, bl1, wo, bo, out_ref):
    h = h_ref[...]
    h = jnp.maximum(jnp.dot(h, w0[...], preferred_element_type=jnp.float32) + b0[...], 0.0)
    h = jnp.maximum(jnp.dot(h, w1[...], preferred_element_type=jnp.float32) + b1[...], 0.0)
    h = jnp.maximum(jnp.dot(h, w2[...], preferred_element_type=jnp.float32) + b2[...], 0.0)
    h = jnp.maximum(jnp.dot(h, wl0[...], preferred_element_type=jnp.float32) + bl0[...], 0.0)
    h = jnp.maximum(jnp.dot(h, wl1[...], preferred_element_type=jnp.float32) + bl1[...], 0.0)
    out_ref[...] = jnp.dot(h, wo[...], preferred_element_type=jnp.float32) + bo[...]


def _head_pallas(h, params):
    n = h.shape[0]
    kin = h.shape[1]
    (w0, b0), (w1, b1), (w2, b2) = params["fp2"]
    (wl0, bl0), (wl1, bl1) = params["lin"]
    (wo, bo) = params["out"][0]
    kp = 136
    h = jnp.pad(h, ((0, 0), (0, kp - kin)))
    w0 = jnp.pad(w0, ((0, kp - kin), (0, 0)))
    wo_p = jnp.pad(wo, ((0, 0), (0, 128 - wo.shape[1])))
    bo_p = jnp.pad(bo, ((0, 128 - bo.shape[0]),))
    blk = 1024
    grid = (n // blk,)
    wspec = lambda shape: pl.BlockSpec(shape, lambda i: (0,) * len(shape))
    out = pl.pallas_call(
        _head_body,
        grid=grid,
        in_specs=[
            pl.BlockSpec((blk, kp), lambda i: (i, 0)),
            wspec((kp, 128)), wspec((128,)),
            wspec((128, 128)), wspec((128,)),
            wspec((128, 128)), wspec((128,)),
            wspec((128, 128)), wspec((128,)),
            wspec((128, 128)), wspec((128,)),
            wspec((128, 128)), wspec((128,)),
        ],
        out_specs=pl.BlockSpec((blk, 128), lambda i: (i, 0)),
        out_shape=jax.ShapeDtypeStruct((n, 128), jnp.float32),
    )(h, w0, b0, w1, b1, w2, b2, wl0, bl0, wl1, bl1, wo_p, bo_p)
    return out[:, :3]


def kernel(x, pos, batch, params):
    pos0 = _fps_pallas(pos, _N, _N0)
    nbr0, v0 = _radius_nbrs(pos, pos0, _R0, _K)
    x0 = _point_conv(params["sa0"], x, pos, pos0, nbr0, v0)
    pos1 = _fps_pallas(pos0, _N0, _N1)
    nbr1, v1 = _radius_nbrs(pos0, pos1, _R1, _K)
    x1 = _point_conv(params["sa1"], x0, pos0, pos1, nbr1, v1)
    g = _mlp(params["sa2"], jnp.concatenate([x1, pos1], axis=1))
    xg = jnp.max(g, axis=0, keepdims=True)
    f0 = jnp.broadcast_to(xg, (_N1, xg.shape[1]))
    f0 = _mlp(params["fp0"], jnp.concatenate([f0, x1], axis=1))
    f1 = _knn3_pallas(f0, pos1, pos0)
    f1 = _mlp(params["fp1"], jnp.concatenate([f1, x0], axis=1))
    f2 = _knn3_pallas(f1, pos0, pos)
    h = jnp.concatenate([f2, x], axis=1)
    return _head_pallas(h, params)


# speedup vs baseline: 9.4424x; 3.4154x over previous
import functools

import jax
import jax.numpy as jnp
import numpy as np
from jax import lax
from jax.experimental import pallas as pl
from jax.experimental.pallas import tpu as pltpu
from jax.experimental.pallas import tpu_sc as plsc

_N = 8192
_N0 = 1638
_N1 = 409
_K = 64
_R0 = 0.2
_R1 = 0.4


def _mlp(ps, x):
    for W, b in ps:
        x = jax.nn.relu(x @ W + b)
    return x


_BIG_I32 = 2**31 - 1


def _fps_body(npoint, n_valid, px_ref, py_ref, pz_ref, ox_ref, oy_ref, oz_ref):
    R, C = px_ref.shape
    OR, OC = ox_ref.shape
    px, py, pz = px_ref[...], py_ref[...], pz_ref[...]
    iota = (lax.broadcasted_iota(jnp.int32, (R, C), 0) * C
            + lax.broadcasted_iota(jnp.int32, (R, C), 1))
    oiota = (lax.broadcasted_iota(jnp.int32, (OR, OC), 0) * OC
             + lax.broadcasted_iota(jnp.int32, (OR, OC), 1))
    valid = iota < n_valid
    x0 = jnp.sum(jnp.where(iota == 0, px, 0.0))
    y0 = jnp.sum(jnp.where(iota == 0, py, 0.0))
    z0 = jnp.sum(jnp.where(iota == 0, pz, 0.0))
    dx, dy, dz = px - x0, py - y0, pz - z0
    d = jnp.where(valid, dx * dx + dy * dy + dz * dz, -1.0)
    ox_ref[...] = jnp.where(oiota == 0, x0, 0.0)
    oy_ref[...] = jnp.where(oiota == 0, y0, 0.0)
    oz_ref[...] = jnp.where(oiota == 0, z0, 0.0)

    def body(i, d):
        m = jnp.max(d)
        nxt = jnp.min(jnp.where(d == m, iota, _BIG_I32))
        sel = iota == nxt
        xn = jnp.sum(jnp.where(sel, px, 0.0))
        yn = jnp.sum(jnp.where(sel, py, 0.0))
        zn = jnp.sum(jnp.where(sel, pz, 0.0))
        dx, dy, dz = px - xn, py - yn, pz - zn
        nd = dx * dx + dy * dy + dz * dz
        om = oiota == i
        ox_ref[...] = jnp.where(om, xn, ox_ref[...])
        oy_ref[...] = jnp.where(om, yn, oy_ref[...])
        oz_ref[...] = jnp.where(om, zn, oz_ref[...])
        return jnp.minimum(d, nd)

    lax.fori_loop(1, npoint, body, d)


def _fps_pallas(pos, n_valid, npoint):
    n_pad = ((n_valid + 127) // 128) * 128
    o_pad = ((npoint + 127) // 128) * 128
    p = jnp.pad(pos, ((0, n_pad - n_valid), (0, 0)))
    px = p[:, 0].reshape(n_pad // 128, 128)
    py = p[:, 1].reshape(n_pad // 128, 128)
    pz = p[:, 2].reshape(n_pad // 128, 128)
    oshape = (o_pad // 128, 128)
    ox, oy, oz = pl.pallas_call(
        functools.partial(_fps_body, npoint, n_valid),
        out_shape=[jax.ShapeDtypeStruct(oshape, jnp.float32)] * 3,
    )(px, py, pz)
    out = jnp.stack([ox.reshape(-1), oy.reshape(-1), oz.reshape(-1)], axis=1)
    return out[:npoint]


_INF_BITS = 0x7F800000


def _rsearch_body(s_valid, r2, q_ref, sx_ref, sy_ref, sz_ref,
                  md2_ref, thr_ref, cnt_ref):
    Qb = q_ref.shape[0]
    Sp = sx_ref.shape[1]
    qx, qy, qz = q_ref[:, 0:1], q_ref[:, 1:2], q_ref[:, 2:3]
    dx = qx - sx_ref[...]
    dy = qy - sy_ref[...]
    dz = qz - sz_ref[...]
    d2 = dx * dx + dy * dy + dz * dz
    jiota = lax.broadcasted_iota(jnp.int32, (Qb, Sp), 1)
    md2 = jnp.where((jiota < s_valid) & (d2 <= r2), d2, jnp.inf)
    md2_ref[...] = md2
    bits = lax.bitcast_convert_type(md2, jnp.int32)
    c_r = jnp.sum((bits < _INF_BITS).astype(jnp.int32), axis=1, keepdims=True)
    cnt_ref[...] = jnp.minimum(c_r, 64)
    lo = jnp.zeros((Qb, 1), jnp.int32)
    hi = jnp.full((Qb, 1), _INF_BITS, jnp.int32)

    def body(_, carry):
        lo, hi = carry
        mid = lo + ((hi - lo) >> 1)
        c = jnp.sum((bits <= mid).astype(jnp.int32), axis=1, keepdims=True)
        small = c < 64
        return (jnp.where(small, mid + 1, lo), jnp.where(small, hi, mid))

    lo, hi = lax.fori_loop(0, 31, body, (lo, hi))
    thr_ref[...] = lax.bitcast_convert_type(
        jnp.where(hi == _INF_BITS, _INF_BITS - 1, hi), jnp.float32)


def _radius_search_pallas(pos_src, s_valid, pos_q, q_valid, r, q_blk):
    S = pos_src.shape[0]
    Sp = ((S + 127) // 128) * 128
    Qp = ((q_valid + q_blk - 1) // q_blk) * q_blk
    sp = jnp.pad(pos_src, ((0, Sp - S), (0, 0)))
    sx = sp[:, 0].reshape(1, Sp)
    sy = sp[:, 1].reshape(1, Sp)
    sz = sp[:, 2].reshape(1, Sp)
    qp = jnp.pad(pos_q, ((0, Qp - q_valid), (0, 0)))
    md2, thr, cnt = pl.pallas_call(
        functools.partial(_rsearch_body, s_valid, r * r),
        grid=(Qp // q_blk,),
        in_specs=[
            pl.BlockSpec((q_blk, 3), lambda i: (i, 0)),
            pl.BlockSpec((1, Sp), lambda i: (0, 0)),
            pl.BlockSpec((1, Sp), lambda i: (0, 0)),
            pl.BlockSpec((1, Sp), lambda i: (0, 0)),
        ],
        out_specs=[
            pl.BlockSpec((q_blk, Sp), lambda i: (i, 0)),
            pl.BlockSpec((q_blk, 1), lambda i: (i, 0)),
            pl.BlockSpec((q_blk, 1), lambda i: (i, 0)),
        ],
        out_shape=[
            jax.ShapeDtypeStruct((Qp, Sp), jnp.float32),
            jax.ShapeDtypeStruct((Qp, 1), jnp.float32),
            jax.ShapeDtypeStruct((Qp, 1), jnp.int32),
        ],
    )(qp, sx, sy, sz)
    return md2, thr, cnt


def _sc_compact(md2, thr, n_per_worker):
    Qp, W = md2.shape
    nv = W // 16
    mesh = plsc.VectorSubcoreMesh(core_axis_name="c", subcore_axis_name="s")

    @functools.partial(
        pl.kernel, mesh=mesh,
        compiler_params=pltpu.CompilerParams(needs_layout_passes=False),
        out_type=jax.ShapeDtypeStruct((Qp, 128), jnp.int32),
        scratch_types=[
            pltpu.VMEM((W,), jnp.float32),
            pltpu.VMEM((W + 16,), jnp.int32),
            pltpu.VMEM((n_per_worker * 16,), jnp.float32),
        ],
    )
    def k(md2_hbm, thr_hbm, out_hbm, row_v, buf_v, thr_v):
        wid = lax.axis_index("s") * 2 + lax.axis_index("c")
        base = wid * n_per_worker
        pltpu.sync_copy(thr_hbm.at[pl.ds(base * 16, n_per_worker * 16)], thr_v)
        zeros16 = jnp.zeros((16,), jnp.int32)
        lane = lax.iota(jnp.int32, 16)

        def per_query(qi, _):
            q = base + qi
            pltpu.sync_copy(md2_hbm.at[q], row_v)
            for sb in range(8):
                buf_v[pl.ds(sb * 16, 16)] = zeros16
            thr_s = thr_v[pl.ds(qi * 16, 16)]

            def per_vreg(vi, cursor):
                v = row_v[pl.ds(vi * 16, 16)]
                take = v <= thr_s
                inc = plsc.cumsum(jnp.where(take, 1, 0))
                pos = inc + cursor - 1
                idxs = lane + vi * 16
                plsc.store_scatter(buf_v, [pos], idxs, mask=take)
                pc = plsc.all_reduce_population_count(take)
                return cursor + pc

            lax.fori_loop(0, nv, per_vreg, zeros16)
            pltpu.sync_copy(buf_v.at[pl.ds(0, 128)], out_hbm.at[q])
            return 0

        lax.fori_loop(0, n_per_worker, per_query, 0)

    return k(md2, thr)


def _radius_nbrs_fast(x_src_pad, pos_src, s_valid, pos_q, q_valid, r, q_blk,
                      n_per_worker):
    md2, thr, cnt = _radius_search_pallas(pos_src, s_valid, pos_q, q_valid, r, q_blk)
    thr_b = jnp.broadcast_to(thr, (thr.shape[0], 16)).reshape(-1)
    nbr = _sc_compact(md2, thr_b, n_per_worker)
    nbr = nbr[:q_valid, :64]
    valid = jnp.arange(64, dtype=jnp.int32)[None, :] < cnt[:q_valid]
    return nbr, valid


def _radius_nbrs(pos_src, pos_q, r, k):
    d2 = jnp.sum((pos_q[:, None, :] - pos_src[None, :, :]) ** 2, axis=-1)
    masked = jnp.where(d2 <= r * r, d2, jnp.inf)
    negv, idx = lax.top_k(-masked, k)
    valid = jnp.isfinite(negv)
    return idx, valid


def _point_conv(ps, x_src, pos_src, pos_q, nbr, valid):
    x_j = x_src[nbr]
    rel = pos_src[nbr] - pos_q[:, None, :]
    h = _mlp(ps, jnp.concatenate([x_j, rel], axis=-1))
    h = jnp.where(valid[:, :, None], h, -jnp.inf)
    out = jnp.max(h, axis=1)
    return jnp.where(jnp.isfinite(out), out, 0.0)


def _knn3_body(s_valid, q_ref, sx_ref, sy_ref, sz_ref, xs_ref, out_ref):
    Qb = q_ref.shape[0]
    Sp = sx_ref.shape[1]
    qx, qy, qz = q_ref[:, 0:1], q_ref[:, 1:2], q_ref[:, 2:3]
    dx = qx - sx_ref[...]
    dy = qy - sy_ref[...]
    dz = qz - sz_ref[...]
    d2 = dx * dx + dy * dy + dz * dz
    jiota = lax.broadcasted_iota(jnp.int32, (Qb, Sp), 1)
    d2 = jnp.where(jiota < s_valid, d2, jnp.inf)
    Wm = jnp.zeros((Qb, Sp), jnp.float32)
    wsum = jnp.zeros((Qb, 1), jnp.float32)
    ws, sels = [], []
    for _ in range(3):
        m = jnp.min(d2, axis=1, keepdims=True)
        i = jnp.min(jnp.where(d2 == m, jiota, _BIG_I32), axis=1, keepdims=True)
        sel = jiota == i
        d2 = jnp.where(sel, jnp.inf, d2)
        w = 1.0 / jnp.maximum(m, 1e-16)
        wsum = wsum + w
        ws.append(w)
        sels.append(sel)
    for w, sel in zip(ws, sels):
        Wm = Wm + jnp.where(sel, w / wsum, 0.0)
    out_ref[...] = jnp.dot(Wm, xs_ref[...], preferred_element_type=jnp.float32)


def _knn3_pallas(x_src, pos_src, pos_dst):
    S, F = x_src.shape
    Q = pos_dst.shape[0]
    Sp = ((S + 127) // 128) * 128
    Qb = 512
    Qp = ((Q + Qb - 1) // Qb) * Qb
    xs = jnp.pad(x_src, ((0, Sp - S), (0, 0)))
    sp = jnp.pad(pos_src, ((0, Sp - S), (0, 0)))
    sx = sp[:, 0].reshape(1, Sp)
    sy = sp[:, 1].reshape(1, Sp)
    sz = sp[:, 2].reshape(1, Sp)
    qp = jnp.pad(pos_dst, ((0, Qp - Q), (0, 0)))
    out = pl.pallas_call(
        functools.partial(_knn3_body, S),
        grid=(Qp // Qb,),
        in_specs=[
            pl.BlockSpec((Qb, 3), lambda i: (i, 0)),
            pl.BlockSpec((1, Sp), lambda i: (0, 0)),
            pl.BlockSpec((1, Sp), lambda i: (0, 0)),
            pl.BlockSpec((1, Sp), lambda i: (0, 0)),
            pl.BlockSpec((Sp, F), lambda i: (0, 0)),
        ],
        out_specs=pl.BlockSpec((Qb, F), lambda i: (i, 0)),
        out_shape=jax.ShapeDtypeStruct((Qp, F), jnp.float32),
    )(qp, sx, sy, sz, xs)
    return out[:Q]


def _knn_interp(x_src, pos_src, pos_dst, k):
    d2 = jnp.sum((pos_dst[:, None, :] - pos_src[None, :, :]) ** 2, axis=-1)
    negv, idx = lax.top_k(-d2, k)
    d2k = jnp.maximum(-negv, 1e-16)
    w = 1.0 / d2k
    w = w / jnp.sum(w, axis=1, keepdims=True)
    return jnp.sum(w[:, :, None] * x_src[idx], axis=1)


def _head_body(h_ref, w0, b0, w1, b1, w2, b2, wl0, bl0, wl1, bl1, wo, bo, out_ref):
    h = h_ref[...]
    h = jnp.maximum(jnp.dot(h, w0[...], preferred_element_type=jnp.float32) + b0[...], 0.0)
    h = jnp.maximum(jnp.dot(h, w1[...], preferred_element_type=jnp.float32) + b1[...], 0.0)
    h = jnp.maximum(jnp.dot(h, w2[...], preferred_element_type=jnp.float32) + b2[...], 0.0)
    h = jnp.maximum(jnp.dot(h, wl0[...], preferred_element_type=jnp.float32) + bl0[...], 0.0)
    h = jnp.maximum(jnp.dot(h, wl1[...], preferred_element_type=jnp.float32) + bl1[...], 0.0)
    out_ref[...] = jnp.dot(h, wo[...], preferred_element_type=jnp.float32) + bo[...]


def _head_pallas(h, params):
    n = h.shape[0]
    kin = h.shape[1]
    (w0, b0), (w1, b1), (w2, b2) = params["fp2"]
    (wl0, bl0), (wl1, bl1) = params["lin"]
    (wo, bo) = params["out"][0]
    kp = 136
    h = jnp.pad(h, ((0, 0), (0, kp - kin)))
    w0 = jnp.pad(w0, ((0, kp - kin), (0, 0)))
    wo_p = jnp.pad(wo, ((0, 0), (0, 128 - wo.shape[1])))
    bo_p = jnp.pad(bo, ((0, 128 - bo.shape[0]),))
    blk = 1024
    grid = (n // blk,)
    wspec = lambda shape: pl.BlockSpec(shape, lambda i: (0,) * len(shape))
    out = pl.pallas_call(
        _head_body,
        grid=grid,
        in_specs=[
            pl.BlockSpec((blk, kp), lambda i: (i, 0)),
            wspec((kp, 128)), wspec((128,)),
            wspec((128, 128)), wspec((128,)),
            wspec((128, 128)), wspec((128,)),
            wspec((128, 128)), wspec((128,)),
            wspec((128, 128)), wspec((128,)),
            wspec((128, 128)), wspec((128,)),
        ],
        out_specs=pl.BlockSpec((blk, 128), lambda i: (i, 0)),
        out_shape=jax.ShapeDtypeStruct((n, 128), jnp.float32),
    )(h, w0, b0, w1, b1, w2, b2, wl0, bl0, wl1, bl1, wo_p, bo_p)
    return out[:, :3]


def kernel(x, pos, batch, params):
    pos0 = _fps_pallas(pos, _N, _N0)
    nbr0, v0 = _radius_nbrs_fast(None, pos, _N, pos0, _N0, _R0,
                                 q_blk=256, n_per_worker=56)
    x0 = _point_conv(params["sa0"], x, pos, pos0, nbr0, v0)
    pos1 = _fps_pallas(pos0, _N0, _N1)
    nbr1, v1 = _radius_nbrs_fast(None, pos0, _N0, pos1, _N1, _R1,
                                 q_blk=256, n_per_worker=16)
    x1 = _point_conv(params["sa1"], x0, pos0, pos1, nbr1, v1)
    g = _mlp(params["sa2"], jnp.concatenate([x1, pos1], axis=1))
    xg = jnp.max(g, axis=0, keepdims=True)
    f0 = jnp.broadcast_to(xg, (_N1, xg.shape[1]))
    f0 = _mlp(params["fp0"], jnp.concatenate([f0, x1], axis=1))
    f1 = _knn3_pallas(f0, pos1, pos0)
    f1 = _mlp(params["fp1"], jnp.concatenate([f1, x0], axis=1))
    f2 = _knn3_pallas(f1, pos0, pos)
    h = jnp.concatenate([f2, x], axis=1)
    return _head_pallas(h, params)


# speedup vs baseline: 10.3128x; 1.0922x over previous
import functools

import jax
import jax.numpy as jnp
import numpy as np
from jax import lax
from jax.experimental import pallas as pl
from jax.experimental.pallas import tpu as pltpu
from jax.experimental.pallas import tpu_sc as plsc

_N = 8192
_N0 = 1638
_N1 = 409
_K = 64
_R0 = 0.2
_R1 = 0.4


def _mlp(ps, x):
    for W, b in ps:
        x = jax.nn.relu(x @ W + b)
    return x


_BIG_I32 = 2**31 - 1


def _fps_body(npoint, n_valid, px_ref, py_ref, pz_ref, ox_ref, oy_ref, oz_ref):
    R, C = px_ref.shape
    OR, OC = ox_ref.shape
    px, py, pz = px_ref[...], py_ref[...], pz_ref[...]
    iota = (lax.broadcasted_iota(jnp.int32, (R, C), 0) * C
            + lax.broadcasted_iota(jnp.int32, (R, C), 1))
    oiota = (lax.broadcasted_iota(jnp.int32, (OR, OC), 0) * OC
             + lax.broadcasted_iota(jnp.int32, (OR, OC), 1))
    valid = iota < n_valid
    x0 = jnp.sum(jnp.where(iota == 0, px, 0.0))
    y0 = jnp.sum(jnp.where(iota == 0, py, 0.0))
    z0 = jnp.sum(jnp.where(iota == 0, pz, 0.0))
    dx, dy, dz = px - x0, py - y0, pz - z0
    d = jnp.where(valid, dx * dx + dy * dy + dz * dz, -1.0)
    ox_ref[...] = jnp.where(oiota == 0, x0, 0.0)
    oy_ref[...] = jnp.where(oiota == 0, y0, 0.0)
    oz_ref[...] = jnp.where(oiota == 0, z0, 0.0)

    def body(i, d):
        m = jnp.max(d)
        nxt = jnp.min(jnp.where(d == m, iota, _BIG_I32))
        sel = iota == nxt
        xn = jnp.sum(jnp.where(sel, px, 0.0))
        yn = jnp.sum(jnp.where(sel, py, 0.0))
        zn = jnp.sum(jnp.where(sel, pz, 0.0))
        dx, dy, dz = px - xn, py - yn, pz - zn
        nd = dx * dx + dy * dy + dz * dz
        om = oiota == i
        ox_ref[...] = jnp.where(om, xn, ox_ref[...])
        oy_ref[...] = jnp.where(om, yn, oy_ref[...])
        oz_ref[...] = jnp.where(om, zn, oz_ref[...])
        return jnp.minimum(d, nd)

    lax.fori_loop(1, npoint, body, d)


def _fps_pallas(pos, n_valid, npoint):
    n_pad = ((n_valid + 127) // 128) * 128
    o_pad = ((npoint + 127) // 128) * 128
    p = jnp.pad(pos, ((0, n_pad - n_valid), (0, 0)))
    px = p[:, 0].reshape(n_pad // 128, 128)
    py = p[:, 1].reshape(n_pad // 128, 128)
    pz = p[:, 2].reshape(n_pad // 128, 128)
    oshape = (o_pad // 128, 128)
    ox, oy, oz = pl.pallas_call(
        functools.partial(_fps_body, npoint, n_valid),
        out_shape=[jax.ShapeDtypeStruct(oshape, jnp.float32)] * 3,
    )(px, py, pz)
    out = jnp.stack([ox.reshape(-1), oy.reshape(-1), oz.reshape(-1)], axis=1)
    return out[:npoint]


_INF_BITS = 0x7F800000


def _rsearch_body(s_valid, r2, q_ref, sx_ref, sy_ref, sz_ref,
                  md2_ref, thr_ref, cnt_ref):
    Qb = q_ref.shape[0]
    Sp = sx_ref.shape[1]
    qx, qy, qz = q_ref[:, 0:1], q_ref[:, 1:2], q_ref[:, 2:3]
    dx = qx - sx_ref[...]
    dy = qy - sy_ref[...]
    dz = qz - sz_ref[...]
    d2 = dx * dx + dy * dy + dz * dz
    jiota = lax.broadcasted_iota(jnp.int32, (Qb, Sp), 1)
    md2 = jnp.where((jiota < s_valid) & (d2 <= r2), d2, jnp.inf)
    md2_ref[...] = md2
    bits = lax.bitcast_convert_type(md2, jnp.int32)
    c_r = jnp.sum((bits < _INF_BITS).astype(jnp.int32), axis=1, keepdims=True)
    cnt_ref[...] = jnp.minimum(c_r, 64)
    lo = jnp.zeros((Qb, 1), jnp.int32)
    hi = jnp.full((Qb, 1), _INF_BITS, jnp.int32)

    def body(_, carry):
        lo, hi = carry
        mid = lo + ((hi - lo) >> 1)
        c = jnp.sum((bits <= mid).astype(jnp.int32), axis=1, keepdims=True)
        small = c < 64
        return (jnp.where(small, mid + 1, lo), jnp.where(small, hi, mid))

    lo, hi = lax.fori_loop(0, 31, body, (lo, hi))
    thr_ref[...] = lax.bitcast_convert_type(
        jnp.where(hi == _INF_BITS, _INF_BITS - 1, hi), jnp.float32)


def _radius_search_pallas(pos_src, s_valid, pos_q, q_valid, r, q_blk):
    S = pos_src.shape[0]
    Sp = ((S + 127) // 128) * 128
    Qp = ((q_valid + q_blk - 1) // q_blk) * q_blk
    sp = jnp.pad(pos_src, ((0, Sp - S), (0, 0)))
    sx = sp[:, 0].reshape(1, Sp)
    sy = sp[:, 1].reshape(1, Sp)
    sz = sp[:, 2].reshape(1, Sp)
    qp = jnp.pad(pos_q, ((0, Qp - q_valid), (0, 0)))
    md2, thr, cnt = pl.pallas_call(
        functools.partial(_rsearch_body, s_valid, r * r),
        grid=(Qp // q_blk,),
        in_specs=[
            pl.BlockSpec((q_blk, 3), lambda i: (i, 0)),
            pl.BlockSpec((1, Sp), lambda i: (0, 0)),
            pl.BlockSpec((1, Sp), lambda i: (0, 0)),
            pl.BlockSpec((1, Sp), lambda i: (0, 0)),
        ],
        out_specs=[
            pl.BlockSpec((q_blk, Sp), lambda i: (i, 0)),
            pl.BlockSpec((q_blk, 1), lambda i: (i, 0)),
            pl.BlockSpec((q_blk, 1), lambda i: (i, 0)),
        ],
        out_shape=[
            jax.ShapeDtypeStruct((Qp, Sp), jnp.float32),
            jax.ShapeDtypeStruct((Qp, 1), jnp.float32),
            jax.ShapeDtypeStruct((Qp, 1), jnp.int32),
        ],
    )(qp, sx, sy, sz)
    return md2, thr, cnt


def _sc_compact(md2, thr, n_per_worker):
    Qp, W = md2.shape
    nv = W // 16
    mesh = plsc.VectorSubcoreMesh(core_axis_name="c", subcore_axis_name="s")

    @functools.partial(
        pl.kernel, mesh=mesh,
        compiler_params=pltpu.CompilerParams(needs_layout_passes=False),
        out_type=jax.ShapeDtypeStruct((Qp, 128), jnp.int32),
        scratch_types=[
            pltpu.VMEM((W,), jnp.float32),
            pltpu.VMEM((W + 16,), jnp.int32),
            pltpu.VMEM((n_per_worker * 16,), jnp.float32),
        ],
    )
    def k(md2_hbm, thr_hbm, out_hbm, row_v, buf_v, thr_v):
        wid = lax.axis_index("s") * 2 + lax.axis_index("c")
        base = wid * n_per_worker
        pltpu.sync_copy(thr_hbm.at[pl.ds(base * 16, n_per_worker * 16)], thr_v)
        zeros16 = jnp.zeros((16,), jnp.int32)
        lane = lax.iota(jnp.int32, 16)

        def per_query(qi, _):
            q = base + qi
            pltpu.sync_copy(md2_hbm.at[q], row_v)
            for sb in range(8):
                buf_v[pl.ds(sb * 16, 16)] = zeros16
            thr_s = thr_v[pl.ds(qi * 16, 16)]

            def per_vreg(vi, cursor):
                takes, incs, pcs = [], [], []
                for s in range(4):
                    v = row_v[pl.ds((vi * 4 + s) * 16, 16)]
                    take = v <= thr_s
                    takes.append(take)
                    incs.append(plsc.cumsum(jnp.where(take, 1, 0)))
                    pcs.append(plsc.all_reduce_population_count(take))
                for s in range(4):
                    pos = incs[s] + cursor - 1
                    idxs = lane + (vi * 4 + s) * 16
                    plsc.store_scatter(buf_v, [pos], idxs, mask=takes[s])
                    cursor = cursor + pcs[s]
                return cursor

            lax.fori_loop(0, nv // 4, per_vreg, zeros16)
            pltpu.sync_copy(buf_v.at[pl.ds(0, 128)], out_hbm.at[q])
            return 0

        lax.fori_loop(0, n_per_worker, per_query, 0)

    return k(md2, thr)


def _radius_nbrs_fast(x_src_pad, pos_src, s_valid, pos_q, q_valid, r, q_blk,
                      n_per_worker):
    md2, thr, cnt = _radius_search_pallas(pos_src, s_valid, pos_q, q_valid, r, q_blk)
    thr_b = jnp.broadcast_to(thr, (thr.shape[0], 16)).reshape(-1)
    nbr = _sc_compact(md2, thr_b, n_per_worker)
    nbr = nbr[:q_valid, :64]
    valid = jnp.arange(64, dtype=jnp.int32)[None, :] < cnt[:q_valid]
    return nbr, valid


def _radius_nbrs(pos_src, pos_q, r, k):
    d2 = jnp.sum((pos_q[:, None, :] - pos_src[None, :, :]) ** 2, axis=-1)
    masked = jnp.where(d2 <= r * r, d2, jnp.inf)
    negv, idx = lax.top_k(-masked, k)
    valid = jnp.isfinite(negv)
    return idx, valid


def _point_conv(ps, x_src, pos_src, pos_q, nbr, valid):
    x_j = x_src[nbr]
    rel = pos_src[nbr] - pos_q[:, None, :]
    h = _mlp(ps, jnp.concatenate([x_j, rel], axis=-1))
    h = jnp.where(valid[:, :, None], h, -jnp.inf)
    out = jnp.max(h, axis=1)
    return jnp.where(jnp.isfinite(out), out, 0.0)


def _knn3_body(s_valid, q_ref, sx_ref, sy_ref, sz_ref, xs_ref, out_ref):
    Qb = q_ref.shape[0]
    Sp = sx_ref.shape[1]
    qx, qy, qz = q_ref[:, 0:1], q_ref[:, 1:2], q_ref[:, 2:3]
    dx = qx - sx_ref[...]
    dy = qy - sy_ref[...]
    dz = qz - sz_ref[...]
    d2 = dx * dx + dy * dy + dz * dz
    jiota = lax.broadcasted_iota(jnp.int32, (Qb, Sp), 1)
    d2 = jnp.where(jiota < s_valid, d2, jnp.inf)
    Wm = jnp.zeros((Qb, Sp), jnp.float32)
    wsum = jnp.zeros((Qb, 1), jnp.float32)
    ws, sels = [], []
    for _ in range(3):
        m = jnp.min(d2, axis=1, keepdims=True)
        i = jnp.min(jnp.where(d2 == m, jiota, _BIG_I32), axis=1, keepdims=True)
        sel = jiota == i
        d2 = jnp.where(sel, jnp.inf, d2)
        w = 1.0 / jnp.maximum(m, 1e-16)
        wsum = wsum + w
        ws.append(w)
        sels.append(sel)
    for w, sel in zip(ws, sels):
        Wm = Wm + jnp.where(sel, w / wsum, 0.0)
    out_ref[...] = jnp.dot(Wm, xs_ref[...], preferred_element_type=jnp.float32)


def _knn3_pallas(x_src, pos_src, pos_dst):
    S, F = x_src.shape
    Q = pos_dst.shape[0]
    Sp = ((S + 127) // 128) * 128
    Qb = 512
    Qp = ((Q + Qb - 1) // Qb) * Qb
    xs = jnp.pad(x_src, ((0, Sp - S), (0, 0)))
    sp = jnp.pad(pos_src, ((0, Sp - S), (0, 0)))
    sx = sp[:, 0].reshape(1, Sp)
    sy = sp[:, 1].reshape(1, Sp)
    sz = sp[:, 2].reshape(1, Sp)
    qp = jnp.pad(pos_dst, ((0, Qp - Q), (0, 0)))
    out = pl.pallas_call(
        functools.partial(_knn3_body, S),
        grid=(Qp // Qb,),
        in_specs=[
            pl.BlockSpec((Qb, 3), lambda i: (i, 0)),
            pl.BlockSpec((1, Sp), lambda i: (0, 0)),
            pl.BlockSpec((1, Sp), lambda i: (0, 0)),
            pl.BlockSpec((1, Sp), lambda i: (0, 0)),
            pl.BlockSpec((Sp, F), lambda i: (0, 0)),
        ],
        out_specs=pl.BlockSpec((Qb, F), lambda i: (i, 0)),
        out_shape=jax.ShapeDtypeStruct((Qp, F), jnp.float32),
    )(qp, sx, sy, sz, xs)
    return out[:Q]


def _knn_interp(x_src, pos_src, pos_dst, k):
    d2 = jnp.sum((pos_dst[:, None, :] - pos_src[None, :, :]) ** 2, axis=-1)
    negv, idx = lax.top_k(-d2, k)
    d2k = jnp.maximum(-negv, 1e-16)
    w = 1.0 / d2k
    w = w / jnp.sum(w, axis=1, keepdims=True)
    return jnp.sum(w[:, :, None] * x_src[idx], axis=1)


def _head_body(h_ref, w0, b0, w1, b1, w2, b2, wl0, bl0, wl1, bl1, wo, bo, out_ref):
    h = h_ref[...]
    h = jnp.maximum(jnp.dot(h, w0[...], preferred_element_type=jnp.float32) + b0[...], 0.0)
    h = jnp.maximum(jnp.dot(h, w1[...], preferred_element_type=jnp.float32) + b1[...], 0.0)
    h = jnp.maximum(jnp.dot(h, w2[...], preferred_element_type=jnp.float32) + b2[...], 0.0)
    h = jnp.maximum(jnp.dot(h, wl0[...], preferred_element_type=jnp.float32) + bl0[...], 0.0)
    h = jnp.maximum(jnp.dot(h, wl1[...], preferred_element_type=jnp.float32) + bl1[...], 0.0)
    out_ref[...] = jnp.dot(h, wo[...], preferred_element_type=jnp.float32) + bo[...]


def _head_pallas(h, params):
    n = h.shape[0]
    kin = h.shape[1]
    (w0, b0), (w1, b1), (w2, b2) = params["fp2"]
    (wl0, bl0), (wl1, bl1) = params["lin"]
    (wo, bo) = params["out"][0]
    kp = 136
    h = jnp.pad(h, ((0, 0), (0, kp - kin)))
    w0 = jnp.pad(w0, ((0, kp - kin), (0, 0)))
    wo_p = jnp.pad(wo, ((0, 0), (0, 128 - wo.shape[1])))
    bo_p = jnp.pad(bo, ((0, 128 - bo.shape[0]),))
    blk = 1024
    grid = (n // blk,)
    wspec = lambda shape: pl.BlockSpec(shape, lambda i: (0,) * len(shape))
    out = pl.pallas_call(
        _head_body,
        grid=grid,
        in_specs=[
            pl.BlockSpec((blk, kp), lambda i: (i, 0)),
            wspec((kp, 128)), wspec((128,)),
            wspec((128, 128)), wspec((128,)),
            wspec((128, 128)), wspec((128,)),
            wspec((128, 128)), wspec((128,)),
            wspec((128, 128)), wspec((128,)),
            wspec((128, 128)), wspec((128,)),
        ],
        out_specs=pl.BlockSpec((blk, 128), lambda i: (i, 0)),
        out_shape=jax.ShapeDtypeStruct((n, 128), jnp.float32),
    )(h, w0, b0, w1, b1, w2, b2, wl0, bl0, wl1, bl1, wo_p, bo_p)
    return out[:, :3]


def kernel(x, pos, batch, params):
    pos0 = _fps_pallas(pos, _N, _N0)
    nbr0, v0 = _radius_nbrs_fast(None, pos, _N, pos0, _N0, _R0,
                                 q_blk=256, n_per_worker=56)
    x0 = _point_conv(params["sa0"], x, pos, pos0, nbr0, v0)
    pos1 = _fps_pallas(pos0, _N0, _N1)
    nbr1, v1 = _radius_nbrs_fast(None, pos0, _N0, pos1, _N1, _R1,
                                 q_blk=256, n_per_worker=16)
    x1 = _point_conv(params["sa1"], x0, pos0, pos1, nbr1, v1)
    g = _mlp(params["sa2"], jnp.concatenate([x1, pos1], axis=1))
    xg = jnp.max(g, axis=0, keepdims=True)
    f0 = jnp.broadcast_to(xg, (_N1, xg.shape[1]))
    f0 = _mlp(params["fp0"], jnp.concatenate([f0, x1], axis=1))
    f1 = _knn3_pallas(f0, pos1, pos0)
    f1 = _mlp(params["fp1"], jnp.concatenate([f1, x0], axis=1))
    f2 = _knn3_pallas(f1, pos0, pos)
    h = jnp.concatenate([f2, x], axis=1)
    return _head_pallas(h, params)
